# baseline (speedup 1.0000x reference)
import jax
import jax.numpy as jnp
from jax.experimental import pallas as pl
from jax.experimental.pallas import tpu as pltpu

_K, _N, _D_IN, _D_OUT = 2, 4096, 256, 256
_BI = 256


def _gcn_block(inp_ref, adj_ref, w_ref, out_ref, s_ref):
    @pl.when(pl.program_id(0) == 0)
    def _compute_support():
        w = w_ref[...].astype(jnp.bfloat16)
        for k in range(_K):
            xk = inp_ref[k].astype(jnp.bfloat16)
            sk = jnp.dot(xk, w, preferred_element_type=jnp.float32)
            s_ref[:, k * _D_OUT:(k + 1) * _D_OUT] = sk.astype(jnp.bfloat16)

    o = jnp.dot(adj_ref[...], s_ref[...], preferred_element_type=jnp.float32)
    o = jnp.maximum(o, 0.0)
    for k in range(_K):
        out_ref[k] = o[:, k * _D_OUT:(k + 1) * _D_OUT]


def kernel(input, adj, weight):
    grid = (_N // _BI,)
    return pl.pallas_call(
        _gcn_block,
        grid=grid,
        in_specs=[
            pl.BlockSpec((_K, _N, _D_IN), lambda i: (0, 0, 0)),
            pl.BlockSpec((_BI, _N), lambda i: (i, 0)),
            pl.BlockSpec((_D_IN, _D_OUT), lambda i: (0, 0)),
        ],
        out_specs=pl.BlockSpec((_K, _BI, _D_OUT), lambda i: (0, i, 0)),
        out_shape=jax.ShapeDtypeStruct((_K, _N, _D_OUT), jnp.float32),
        scratch_shapes=[pltpu.VMEM((_N, _K * _D_OUT), jnp.bfloat16)],
        compiler_params=pltpu.CompilerParams(
            dimension_semantics=("arbitrary",),
        ),
    )(input, adj, weight)


# speedup vs baseline: 1.1363x; 1.1363x over previous
import jax
import jax.numpy as jnp
from jax.experimental import pallas as pl
from jax.experimental.pallas import tpu as pltpu

_K, _N, _D_IN, _D_OUT = 2, 4096, 256, 256
_BI = 1024


def _gcn_block(inp_ref, adj_ref, w_ref, out_ref, s_ref):
    @pl.when(pl.program_id(0) == 0)
    def _compute_support():
        w = w_ref[...].astype(jnp.bfloat16)
        for k in range(_K):
            xk = inp_ref[k].astype(jnp.bfloat16)
            sk = jnp.dot(xk, w, preferred_element_type=jnp.float32)
            s_ref[:, k * _D_OUT:(k + 1) * _D_OUT] = sk.astype(jnp.bfloat16)

    o = jnp.dot(adj_ref[...], s_ref[...], preferred_element_type=jnp.float32)
    o = jnp.maximum(o, 0.0)
    for k in range(_K):
        out_ref[k] = o[:, k * _D_OUT:(k + 1) * _D_OUT]


def kernel(input, adj, weight):
    grid = (_N // _BI,)
    return pl.pallas_call(
        _gcn_block,
        grid=grid,
        in_specs=[
            pl.BlockSpec((_K, _N, _D_IN), lambda i: (0, 0, 0)),
            pl.BlockSpec((_BI, _N), lambda i: (i, 0)),
            pl.BlockSpec((_D_IN, _D_OUT), lambda i: (0, 0)),
        ],
        out_specs=pl.BlockSpec((_K, _BI, _D_OUT), lambda i: (0, i, 0)),
        out_shape=jax.ShapeDtypeStruct((_K, _N, _D_OUT), jnp.float32),
        scratch_shapes=[pltpu.VMEM((_N, _K * _D_OUT), jnp.bfloat16)],
        compiler_params=pltpu.CompilerParams(
            dimension_semantics=("arbitrary",),
        ),
    )(input, adj, weight)
